# baseline (speedup 1.0000x reference)
import functools

import jax
import jax.numpy as jnp
from jax import lax
from jax.experimental import pallas as pl
from jax.experimental.pallas import tpu as pltpu
from jax.experimental.pallas import tpu_sc as plsc

B = 16384
NC = 2
NS = 16
NW = NC * NS
BPW = B // NW
L = 16
RB = 128

SMALL = [(3, 2), (11, 1), (6, 3), (17, 16), (9, 8), (9, 8)]
SMALL_OFF = []
_o = 0
for _v, _d in SMALL:
    SMALL_OFF.append(_o)
    _o += _v * _d
SMALL_TOT = _o
SMALL_PAD = (SMALL_TOT + 7) // 8 * 8

W1_BIG = [0, 4, 8]
W1_SMALL = [12, 14, 15, 18, 34, 42]
W1_TIME = 50
W1_WEIGHT = 51

MLP_W1 = 0
MLP_B1 = 52 * 8
MLP_W2 = MLP_B1 + 8
MLP_B2 = MLP_W2 + 8
MLP_TOT = MLP_B2 + 1
MLP_PAD = (MLP_TOT + 15) // 16 * 16

PT_STRIDE = 32 * 8


def _scalars(ref, start, n):
    out = []
    for c in range(0, n, L):
        v = ref[pl.ds(start + c, L)]
        out.extend(v[j] for j in range(min(L, n - c)))
    return out


def _sc_body(idx3, den3, ut, pt_, mt, sm, mlp, out3,
             idxv, denv, urows, prows, mrows, smv, ptv, mlpv, outv, sem):
    cid = lax.axis_index("c")
    sid = lax.axis_index("s")
    wid = sid * NC + cid
    rb = wid * 4

    pltpu.sync_copy(idx3.at[:, pl.ds(rb, 4), :], idxv)
    pltpu.sync_copy(den3.at[:, pl.ds(rb, 4), :], denv)
    pltpu.sync_copy(sm, smv)
    pltpu.sync_copy(mlp, mlpv)

    copies = []
    for t, (tab, dst) in enumerate(((ut, urows), (pt_, prows), (mt, mrows))):
        for i in range(4):
            copies.append(
                pltpu.async_copy(tab.at[idxv.at[t, i]],
                                 dst.at[pl.ds(i * RB, RB)], sem))

    iota = lax.iota(jnp.int32, L)

    for ti, (v, d) in enumerate(SMALL):
        base = SMALL_OFF[ti]
        w1s = _scalars(mlpv, MLP_W1 + W1_SMALL[ti] * 8, d * 8)
        for ch in range((v + L - 1) // L):
            rows = jnp.minimum(iota + ch * L, v - 1)
            cols = [plsc.load_gather(smv, [rows * d + (base + dd)])
                    for dd in range(d)]
            prow = (iota + ch * L) * 8 + ti * PT_STRIDE
            for k in range(8):
                acc = cols[0] * w1s[k]
                for dd in range(1, d):
                    acc = acc + cols[dd] * w1s[dd * 8 + k]
                plsc.store_scatter(ptv, [prow + k], acc)

    for cp in copies:
        cp.wait()

    w1big_flat = _scalars(mlpv, MLP_W1, 12 * 8)
    w1big = [w1big_flat[j * 8:(j + 1) * 8] for j in range(12)]
    w1t = _scalars(mlpv, MLP_W1 + W1_TIME * 8, 8)
    w1w = _scalars(mlpv, MLP_W1 + W1_WEIGHT * 8, 8)
    b1 = _scalars(mlpv, MLP_B1, 8)
    w2 = _scalars(mlpv, MLP_W2, 8)
    b2 = _scalars(mlpv, MLP_B2, 1)[0]

    for r in range(4):
        def chunk(cc, carry, r=r):
            o = pl.multiple_of(cc * L, L)
            gbase = r * RB
            ridx = iota + (gbase + o)

            tcol = denv[0, r, pl.ds(o, L)]
            wcol = denv[1, r, pl.ds(o, L)]
            hs = [tcol * w1t[k] + wcol * w1w[k] + b1[k] for k in range(8)]

            for t, rowsbuf in enumerate((urows, prows, mrows)):
                for dd in range(4):
                    col = plsc.load_gather(
                        rowsbuf, [ridx, jnp.full((L,), dd, jnp.int32)])
                    ws = w1big[t * 4 + dd]
                    for k in range(8):
                        hs[k] = hs[k] + col * ws[k]

            for ti in range(6):
                i16 = idxv[3 + ti, r, pl.ds(o, L)]
                fbase = i16 * 8 + ti * PT_STRIDE
                for k in range(8):
                    hs[k] = hs[k] + plsc.load_gather(ptv, [fbase + k])

            z = jnp.maximum(hs[0], 0.0) * w2[0]
            for k in range(1, 8):
                z = z + jnp.maximum(hs[k], 0.0) * w2[k]
            res = 1.0 / (1.0 + jnp.exp(-(z + b2)))
            outv[r, pl.ds(o, L)] = res
            return carry

        lax.fori_loop(0, RB // L, chunk, 0)

    pltpu.sync_copy(outv, out3.at[pl.ds(rb, 4)])


@jax.jit
def _run(idx3, den3, user_tab, product_tab, model_tab, sm, mlp):
    mesh = plsc.VectorSubcoreMesh(core_axis_name="c", subcore_axis_name="s")
    f = functools.partial(
        pl.kernel, mesh=mesh,
        compiler_params=pltpu.CompilerParams(
            needs_layout_passes=False, use_tc_tiling_on_sc=False),
        out_type=jax.ShapeDtypeStruct((128, 128), jnp.float32),
        scratch_types=[
            pltpu.VMEM((9, 4, RB), jnp.int32),
            pltpu.VMEM((2, 4, RB), jnp.float32),
            pltpu.VMEM((BPW, 4), jnp.float32),
            pltpu.VMEM((BPW, 4), jnp.float32),
            pltpu.VMEM((BPW, 4), jnp.float32),
            pltpu.VMEM((SMALL_PAD,), jnp.float32),
            pltpu.VMEM((6 * PT_STRIDE,), jnp.float32),
            pltpu.VMEM((MLP_PAD,), jnp.float32),
            pltpu.VMEM((4, RB), jnp.float32),
            pltpu.SemaphoreType.DMA,
        ],
    )(_sc_body)
    return f(idx3, den3, user_tab, product_tab, model_tab, sm, mlp)


def kernel(x, user_tab, product_tab, model_tab, gender_tab, age_tab, res_tab,
           color_tab, size_tab, mat_tab, W1, b1, W2, b2):
    idx3 = x[:, :9].astype(jnp.int32).T.reshape(9, 128, 128)
    den3 = x[:, 9:11].T.reshape(2, 128, 128)
    sm = jnp.concatenate([
        gender_tab.reshape(-1), age_tab.reshape(-1), res_tab.reshape(-1),
        color_tab.reshape(-1), size_tab.reshape(-1), mat_tab.reshape(-1),
        jnp.zeros((SMALL_PAD - SMALL_TOT,), jnp.float32)])
    mlp = jnp.concatenate([
        W1.reshape(-1), b1.reshape(-1), W2.reshape(-1), b2.reshape(-1),
        jnp.zeros((MLP_PAD - MLP_TOT,), jnp.float32)])
    out = _run(idx3, den3, user_tab, product_tab, model_tab, sm, mlp)
    return out.reshape(B)


# speedup vs baseline: 12.5405x; 12.5405x over previous
import functools

import jax
import jax.numpy as jnp
from jax import lax
from jax.experimental import pallas as pl
from jax.experimental.pallas import tpu as pltpu
from jax.experimental.pallas import tpu_sc as plsc

B = 16384
NC = 2
NS = 16
NW = NC * NS
BPW = B // NW
L = 16
TR = 128
NT = BPW // TR

SMALL = [(3, 2), (11, 1), (6, 3), (17, 16), (9, 8), (9, 8)]
SMALL_OFF = []
_o = 0
for _v, _d in SMALL:
    SMALL_OFF.append(_o)
    _o += _v * _d
SMALL_TOT = _o
SMALL_PAD = (SMALL_TOT + 15) // 16 * 16

W1_BIG = [0, 4, 8]
W1_SMALL = [12, 14, 15, 18, 34, 42]
W1_TIME = 50
W1_WEIGHT = 51

MLP_W1 = 0
MLP_B1 = 52 * 8
MLP_W2 = MLP_B1 + 8
MLP_B2 = MLP_W2 + 8
MLP_TOT = MLP_B2 + 1
MLP_PAD = (MLP_TOT + 15) // 16 * 16

PT_STRIDE = 32 * 8


def _scalars(ref, start, n):
    out = []
    for c in range(0, n, L):
        v = ref[pl.ds(start + c, L)]
        out.extend(v[j] for j in range(min(L, n - c)))
    return out


def _sc_body(idxf, denf,
             uc0, uc1, uc2, uc3, pc0, pc1, pc2, pc3, mc0, mc1, mc2, mc3,
             sm, mlp, outf,
             idxv, ebig, denv, ucols, pcols, mcols, smv, ptv, mlpv, outv, sem):
    cid = lax.axis_index("c")
    sid = lax.axis_index("s")
    wid = sid * NC + cid
    base = wid * BPW

    for j in range(9):
        pltpu.sync_copy(idxf.at[pl.ds(j * B + base, BPW)], idxv.at[j])
    for t in range(3):
        for i in range(NT):
            pltpu.sync_copy(idxf.at[pl.ds(t * B + base + i * TR, TR)],
                            ebig.at[t * NT + i])
    for j in range(2):
        pltpu.sync_copy(denf.at[pl.ds(j * B + base, BPW)], denv.at[j])
    pltpu.sync_copy(sm, smv)
    pltpu.sync_copy(mlp, mlpv)

    cols_hbm = ((uc0, uc1, uc2, uc3), (pc0, pc1, pc2, pc3),
                (mc0, mc1, mc2, mc3))
    rounds = []
    for i in range(NT):
        grp = []
        for t, dst in enumerate((ucols, pcols, mcols)):
            idx_ref = ebig.at[t * NT + i]
            for d in range(4):
                grp.append(
                    pltpu.async_copy(cols_hbm[t][d].at[idx_ref],
                                     dst.at[d, pl.ds(i * TR, TR)], sem))
        rounds.append(grp)
        if i >= 1:
            for cp in rounds[i - 1]:
                cp.wait()

    iota = lax.iota(jnp.int32, L)

    for ti, (v, d) in enumerate(SMALL):
        off = SMALL_OFF[ti]
        w1s = _scalars(mlpv, MLP_W1 + W1_SMALL[ti] * 8, d * 8)
        for ch in range((v + L - 1) // L):
            rows = jnp.minimum(iota + ch * L, v - 1)
            cols = [plsc.load_gather(smv, [rows * d + (off + dd)])
                    for dd in range(d)]
            prow = (iota + ch * L) * 8 + ti * PT_STRIDE
            for k in range(8):
                acc = cols[0] * w1s[k]
                for dd in range(1, d):
                    acc = acc + cols[dd] * w1s[dd * 8 + k]
                plsc.store_scatter(ptv, [prow + k], acc)

    w1big_flat = _scalars(mlpv, MLP_W1, 12 * 8)
    w1big = [w1big_flat[j * 8:(j + 1) * 8] for j in range(12)]
    w1t = _scalars(mlpv, MLP_W1 + W1_TIME * 8, 8)
    w1w = _scalars(mlpv, MLP_W1 + W1_WEIGHT * 8, 8)
    b1 = _scalars(mlpv, MLP_B1, 8)
    w2 = _scalars(mlpv, MLP_W2, 8)
    b2 = _scalars(mlpv, MLP_B2, 1)[0]

    for cp in rounds[NT - 1]:
        cp.wait()

    for r in range(NT):
        def chunk(cc, carry, r=r):
            o = pl.multiple_of(r * TR + cc * L, L)

            tcol = denv[0, pl.ds(o, L)]
            wcol = denv[1, pl.ds(o, L)]
            hs = [tcol * w1t[k] + wcol * w1w[k] + b1[k] for k in range(8)]

            for t, cbuf in enumerate((ucols, pcols, mcols)):
                for dd in range(4):
                    col = cbuf[dd, pl.ds(o, L)]
                    ws = w1big[t * 4 + dd]
                    for k in range(8):
                        hs[k] = hs[k] + col * ws[k]

            for ti in range(6):
                i16 = idxv[3 + ti, pl.ds(o, L)]
                fbase = i16 * 8 + ti * PT_STRIDE
                for k in range(8):
                    hs[k] = hs[k] + plsc.load_gather(ptv, [fbase + k])

            z = jnp.maximum(hs[0], 0.0) * w2[0]
            for k in range(1, 8):
                z = z + jnp.maximum(hs[k], 0.0) * w2[k]
            res = 1.0 / (1.0 + jnp.exp(-(z + b2)))
            outv[pl.ds(o, L)] = res
            return carry

        lax.fori_loop(0, TR // L, chunk, 0)

    pltpu.sync_copy(outv, outf.at[pl.ds(base, BPW)])


@jax.jit
def _run(idxf, denf, ucs, pcs, mcs, sm, mlp):
    mesh = plsc.VectorSubcoreMesh(core_axis_name="c", subcore_axis_name="s")
    f = functools.partial(
        pl.kernel, mesh=mesh,
        compiler_params=pltpu.CompilerParams(
            needs_layout_passes=False, use_tc_tiling_on_sc=False),
        out_type=jax.ShapeDtypeStruct((B,), jnp.float32),
        scratch_types=[
            pltpu.VMEM((9, BPW), jnp.int32),
            pltpu.VMEM((12, TR), jnp.int32),
            pltpu.VMEM((2, BPW), jnp.float32),
            pltpu.VMEM((4, BPW), jnp.float32),
            pltpu.VMEM((4, BPW), jnp.float32),
            pltpu.VMEM((4, BPW), jnp.float32),
            pltpu.VMEM((SMALL_PAD,), jnp.float32),
            pltpu.VMEM((6 * PT_STRIDE,), jnp.float32),
            pltpu.VMEM((MLP_PAD,), jnp.float32),
            pltpu.VMEM((BPW,), jnp.float32),
            pltpu.SemaphoreType.DMA,
        ],
    )(_sc_body)
    return f(idxf, denf, *ucs, *pcs, *mcs, sm, mlp)


def kernel(x, user_tab, product_tab, model_tab, gender_tab, age_tab, res_tab,
           color_tab, size_tab, mat_tab, W1, b1, W2, b2):
    idxf = x[:, :9].astype(jnp.int32).T.reshape(-1)
    denf = x[:, 9:11].T.reshape(-1)
    ucs = [user_tab[:, d] for d in range(4)]
    pcs = [product_tab[:, d] for d in range(4)]
    mcs = [model_tab[:, d] for d in range(4)]
    sm = jnp.concatenate([
        gender_tab.reshape(-1), age_tab.reshape(-1), res_tab.reshape(-1),
        color_tab.reshape(-1), size_tab.reshape(-1), mat_tab.reshape(-1),
        jnp.zeros((SMALL_PAD - SMALL_TOT,), jnp.float32)])
    mlp = jnp.concatenate([
        W1.reshape(-1), b1.reshape(-1), W2.reshape(-1), b2.reshape(-1),
        jnp.zeros((MLP_PAD - MLP_TOT,), jnp.float32)])
    return _run(idxf, denf, ucs, pcs, mcs, sm, mlp)


# speedup vs baseline: 13.5271x; 1.0787x over previous
import functools

import jax
import jax.numpy as jnp
from jax import lax
from jax.experimental import pallas as pl
from jax.experimental.pallas import tpu as pltpu
from jax.experimental.pallas import tpu_sc as plsc

B = 16384
NC = 2
NS = 16
NW = NC * NS
BPW = B // NW
L = 16
TR = 128
NT = BPW // TR

SMALL = [(3, 2), (11, 1), (6, 3), (17, 16), (9, 8), (9, 8)]
SMALL_OFF = []
_o = 0
for _v, _d in SMALL:
    SMALL_OFF.append(_o)
    _o += _v * _d
SMALL_TOT = _o
SMALL_PAD = (SMALL_TOT + 15) // 16 * 16

W1_BIG = [0, 4, 8]
W1_SMALL = [12, 14, 15, 18, 34, 42]
W1_TIME = 50
W1_WEIGHT = 51

MLP_W1 = 0
MLP_B1 = 52 * 8
MLP_W2 = MLP_B1 + 8
MLP_B2 = MLP_W2 + 8
MLP_TOT = MLP_B2 + 1
MLP_PAD = (MLP_TOT + 15) // 16 * 16

PT_STRIDE = 32 * 8


def _scalars(ref, start, n):
    out = []
    for c in range(0, n, L):
        v = ref[pl.ds(start + c, L)]
        out.extend(v[j] for j in range(min(L, n - c)))
    return out


def _sc_body(idxf, denf,
             uc0, uc1, uc2, uc3, pc0, pc1, pc2, pc3, mc0, mc1, mc2, mc3,
             sm, mlp, outf,
             idxv, ebig, denv, ucols, pcols, mcols, smv, ptv, mlpv, outv, sem):
    cid = lax.axis_index("c")
    sid = lax.axis_index("s")
    wid = sid * NC + cid
    base = wid * BPW

    stage = []
    for t in range(3):
        for i in range(NT):
            stage.append(
                pltpu.async_copy(idxf.at[pl.ds(t * B + base + i * TR, TR)],
                                 ebig.at[t * NT + i], sem))
    for j in range(9):
        stage.append(
            pltpu.async_copy(idxf.at[pl.ds(j * B + base, BPW)], idxv.at[j],
                             sem))
    for j in range(2):
        stage.append(
            pltpu.async_copy(denf.at[pl.ds(j * B + base, BPW)], denv.at[j],
                             sem))
    stage.append(pltpu.async_copy(sm, smv, sem))
    stage.append(pltpu.async_copy(mlp, mlpv, sem))
    for cp in stage:
        cp.wait()

    cols_hbm = ((uc0, uc1, uc2, uc3), (pc0, pc1, pc2, pc3),
                (mc0, mc1, mc2, mc3))
    copies = []
    for t, dst in enumerate((ucols, pcols, mcols)):
        for i in range(NT):
            idx_ref = ebig.at[t * NT + i]
            for d in range(4):
                copies.append(
                    pltpu.async_copy(cols_hbm[t][d].at[idx_ref],
                                     dst.at[d, pl.ds(i * TR, TR)], sem))

    iota = lax.iota(jnp.int32, L)

    for ti, (v, d) in enumerate(SMALL):
        off = SMALL_OFF[ti]
        w1s = _scalars(mlpv, MLP_W1 + W1_SMALL[ti] * 8, d * 8)
        for ch in range((v + L - 1) // L):
            rows = jnp.minimum(iota + ch * L, v - 1)
            cols = [plsc.load_gather(smv, [rows * d + (off + dd)])
                    for dd in range(d)]
            prow = (iota + ch * L) * 8 + ti * PT_STRIDE
            for k in range(8):
                acc = cols[0] * w1s[k]
                for dd in range(1, d):
                    acc = acc + cols[dd] * w1s[dd * 8 + k]
                plsc.store_scatter(ptv, [prow + k], acc)

    w1big_flat = _scalars(mlpv, MLP_W1, 12 * 8)
    w1big = [w1big_flat[j * 8:(j + 1) * 8] for j in range(12)]
    w1t = _scalars(mlpv, MLP_W1 + W1_TIME * 8, 8)
    w1w = _scalars(mlpv, MLP_W1 + W1_WEIGHT * 8, 8)
    b1 = _scalars(mlpv, MLP_B1, 8)
    w2 = _scalars(mlpv, MLP_W2, 8)
    b2 = _scalars(mlpv, MLP_B2, 1)[0]

    for cp in copies:
        cp.wait()

    for r in range(NT):
        def chunk(cc, carry, r=r):
            o = pl.multiple_of(r * TR + cc * L, L)

            tcol = denv[0, pl.ds(o, L)]
            wcol = denv[1, pl.ds(o, L)]
            hs = [tcol * w1t[k] + wcol * w1w[k] + b1[k] for k in range(8)]

            for t, cbuf in enumerate((ucols, pcols, mcols)):
                for dd in range(4):
                    col = cbuf[dd, pl.ds(o, L)]
                    ws = w1big[t * 4 + dd]
                    for k in range(8):
                        hs[k] = hs[k] + col * ws[k]

            for ti in range(6):
                i16 = idxv[3 + ti, pl.ds(o, L)]
                fbase = i16 * 8 + ti * PT_STRIDE
                for k in range(8):
                    hs[k] = hs[k] + plsc.load_gather(ptv, [fbase + k])

            z = jnp.maximum(hs[0], 0.0) * w2[0]
            for k in range(1, 8):
                z = z + jnp.maximum(hs[k], 0.0) * w2[k]
            res = 1.0 / (1.0 + jnp.exp(-(z + b2)))
            outv[pl.ds(o, L)] = res
            return carry

        lax.fori_loop(0, TR // L, chunk, 0)

    pltpu.sync_copy(outv, outf.at[pl.ds(base, BPW)])


@jax.jit
def _run(idxf, denf, ucs, pcs, mcs, sm, mlp):
    mesh = plsc.VectorSubcoreMesh(core_axis_name="c", subcore_axis_name="s")
    f = functools.partial(
        pl.kernel, mesh=mesh,
        compiler_params=pltpu.CompilerParams(
            needs_layout_passes=False, use_tc_tiling_on_sc=False),
        out_type=jax.ShapeDtypeStruct((B,), jnp.float32),
        scratch_types=[
            pltpu.VMEM((9, BPW), jnp.int32),
            pltpu.VMEM((12, TR), jnp.int32),
            pltpu.VMEM((2, BPW), jnp.float32),
            pltpu.VMEM((4, BPW), jnp.float32),
            pltpu.VMEM((4, BPW), jnp.float32),
            pltpu.VMEM((4, BPW), jnp.float32),
            pltpu.VMEM((SMALL_PAD,), jnp.float32),
            pltpu.VMEM((6 * PT_STRIDE,), jnp.float32),
            pltpu.VMEM((MLP_PAD,), jnp.float32),
            pltpu.VMEM((BPW,), jnp.float32),
            pltpu.SemaphoreType.DMA,
        ],
    )(_sc_body)
    return f(idxf, denf, *ucs, *pcs, *mcs, sm, mlp)


def kernel(x, user_tab, product_tab, model_tab, gender_tab, age_tab, res_tab,
           color_tab, size_tab, mat_tab, W1, b1, W2, b2):
    idxf = x[:, :9].astype(jnp.int32).T.reshape(-1)
    denf = x[:, 9:11].T.reshape(-1)
    ucs = [user_tab[:, d] for d in range(4)]
    pcs = [product_tab[:, d] for d in range(4)]
    mcs = [model_tab[:, d] for d in range(4)]
    sm = jnp.concatenate([
        gender_tab.reshape(-1), age_tab.reshape(-1), res_tab.reshape(-1),
        color_tab.reshape(-1), size_tab.reshape(-1), mat_tab.reshape(-1),
        jnp.zeros((SMALL_PAD - SMALL_TOT,), jnp.float32)])
    mlp = jnp.concatenate([
        W1.reshape(-1), b1.reshape(-1), W2.reshape(-1), b2.reshape(-1),
        jnp.zeros((MLP_PAD - MLP_TOT,), jnp.float32)])
    return _run(idxf, denf, ucs, pcs, mcs, sm, mlp)
